# Initial kernel scaffold; baseline (speedup 1.0000x reference)
import jax
import jax.numpy as jnp
from jax.experimental import pallas as pl


def kernel(x, members, W_ode, b_ode, W_conv, b_conv):
    raise NotImplementedError("write your pallas kernel here")


# speedup vs baseline: 12.4987x; 12.4987x over previous
import functools

import jax
import jax.numpy as jnp
from jax import lax
from jax.experimental import pallas as pl
from jax.experimental.pallas import tpu as pltpu
from jax.experimental.pallas import tpu_sc as plsc

NC, NS, LN = 2, 16, 16
N = 10000
NPAD = 10240
M = 80000
MPAD = 81920
D = 128
CW = D // NC
S = 4
CF = 128
FPT = MPAD // NS
NCHUNK = FPT // CF
ROWS_PT = NPAD // NS
RCHUNK = ROWS_PT // CF
DT = 0.25

_mesh = plsc.VectorSubcoreMesh(core_axis_name="c", subcore_axis_name="s")


@functools.partial(
    pl.kernel,
    out_type=jax.ShapeDtypeStruct((NC, NPAD, CW), jnp.float32),
    mesh=_mesh,
    scratch_types=[
        pltpu.VMEM((CF,), jnp.int32),
        pltpu.VMEM((CF,), jnp.int32),
        pltpu.VMEM((CF,), jnp.int32),
        pltpu.VMEM((CF,), jnp.int32),
        pltpu.VMEM((CF, CW), jnp.float32),
        pltpu.VMEM((CF, CW), jnp.float32),
        pltpu.VMEM((CF, CW), jnp.float32),
        pltpu.VMEM((CF, CW), jnp.float32),
        pltpu.VMEM((CF, CW), jnp.float32),
        pltpu.VMEM_SHARED((NPAD, CW), jnp.float32),
        pltpu.SemaphoreType.DMA,
    ],
    compiler_params=pltpu.CompilerParams(use_tc_tiling_on_sc=False),
)
def _agg(h_hbm, m0, m1, m2, m3, out_hbm,
         idx0, idx1, idx2, idx3, r0, r1, r2, r3, ebuf, acc, sem):
    c = lax.axis_index("c")
    s = lax.axis_index("s")
    fbase = s * FPT
    rbase = s * ROWS_PT
    hc = h_hbm.at[c]

    z = jnp.zeros((LN,), jnp.float32)

    def zrow(i, _):
        for g in range(CW // LN):
            ebuf[i, pl.ds(g * LN, LN)] = z
        return 0

    lax.fori_loop(0, CF, zrow, 0, unroll=False)
    for k in range(RCHUNK):
        pltpu.sync_copy(ebuf, acc.at[pl.ds(rbase + k * CF, CF)])
    plsc.subcore_barrier()

    def chunk(k, _):
        off = pl.multiple_of(fbase + k * CF, CF)
        pltpu.sync_copy(m0.at[pl.ds(off, CF)], idx0)
        pltpu.sync_copy(m1.at[pl.ds(off, CF)], idx1)
        pltpu.sync_copy(m2.at[pl.ds(off, CF)], idx2)
        pltpu.sync_copy(m3.at[pl.ds(off, CF)], idx3)
        cp0 = pltpu.async_copy(hc.at[idx0], r0, sem)
        cp1 = pltpu.async_copy(hc.at[idx1], r1, sem)
        cp2 = pltpu.async_copy(hc.at[idx2], r2, sem)
        cp3 = pltpu.async_copy(hc.at[idx3], r3, sem)
        cp0.wait()
        cp1.wait()
        cp2.wait()
        cp3.wait()

        def row(i, _):
            for g in range(CW // LN):
                sl = pl.ds(g * LN, LN)
                ebuf[i, sl] = (r0[i, sl] + r1[i, sl]) + (r2[i, sl] + r3[i, sl])
            return 0

        lax.fori_loop(0, CF, row, 0, unroll=False)
        pltpu.sync_copy(ebuf, acc.at[idx0], add=True)
        pltpu.sync_copy(ebuf, acc.at[idx1], add=True)
        pltpu.sync_copy(ebuf, acc.at[idx2], add=True)
        pltpu.sync_copy(ebuf, acc.at[idx3], add=True)
        return 0

    lax.fori_loop(0, NCHUNK, chunk, 0, unroll=False)
    plsc.subcore_barrier()

    for k in range(RCHUNK):
        r = rbase + k * CF
        pltpu.sync_copy(acc.at[pl.ds(r, CF)], ebuf)
        pltpu.sync_copy(ebuf, out_hbm.at[c].at[pl.ds(r, CF)])


def _dinv_body(degp_ref, o_ref):
    deg = degp_ref[0, :, 0] * 0.25
    o_ref[...] = (0.25 / jnp.maximum(deg, 1.0))[:, None]


def _step_body(p_ref, dinv_ref, h_ref, w_ref, b_ref, o_ref):
    a = jnp.concatenate([p_ref[0], p_ref[1]], axis=1) * dinv_ref[...]
    z = jnp.dot(a, w_ref[...], preferred_element_type=jnp.float32) + b_ref[...]
    u = DT * jnp.tanh(z)
    o_ref[0] = h_ref[0] + u[:, :CW]
    o_ref[1] = h_ref[1] + u[:, CW:]


def _final_body(p_ref, dinv_ref, wc_ref, bc_ref, marg_ref, map_ref):
    a = jnp.concatenate([p_ref[0], p_ref[1]], axis=1) * dinv_ref[...]
    logits = jnp.dot(a, wc_ref[...], preferred_element_type=jnp.float32) + bc_ref[...]
    mx = jnp.max(logits, axis=-1, keepdims=True)
    sh = logits - mx
    ev = sh - jnp.log(jnp.sum(jnp.exp(sh), axis=-1, keepdims=True))
    mx2 = jnp.max(ev, axis=-1, keepdims=True)
    ex = jnp.exp(ev - mx2)
    marg_ref[...] = ex / jnp.sum(ex, axis=-1, keepdims=True)
    iot = lax.broadcasted_iota(jnp.int32, ev.shape, 1)
    cand = jnp.where(ev >= mx2, iot, S)
    map_ref[...] = jnp.min(cand, axis=-1, keepdims=True)


_dinv = pl.pallas_call(
    _dinv_body,
    out_shape=jax.ShapeDtypeStruct((NPAD, 1), jnp.float32),
)

_step = pl.pallas_call(
    _step_body,
    out_shape=jax.ShapeDtypeStruct((NC, NPAD, CW), jnp.float32),
)

_final = pl.pallas_call(
    _final_body,
    out_shape=(
        jax.ShapeDtypeStruct((NPAD, S), jnp.float32),
        jax.ShapeDtypeStruct((NPAD, 1), jnp.int32),
    ),
)


def kernel(x, members, W_ode, b_ode, W_conv, b_conv):
    mT = members.T
    pad = jnp.full((4, MPAD - M), N, jnp.int32)
    mcols = jnp.concatenate([mT, pad], axis=1)
    m0, m1, m2, m3 = mcols[0], mcols[1], mcols[2], mcols[3]
    xp = jnp.pad(x, ((0, NPAD - N), (0, 0)))
    h = xp.reshape(NPAD, NC, CW).transpose(1, 0, 2)

    ones_tab = jnp.ones((NC, NPAD, CW), jnp.float32)
    degp = _agg(ones_tab, m0, m1, m2, m3)
    dinv = _dinv(degp)
    wb = b_ode[None, :]
    for _ in range(4):
        p = _agg(h, m0, m1, m2, m3)
        h = _step(p, dinv, h, W_ode, wb)
    p = _agg(h, m0, m1, m2, m3)
    marg, mp = _final(p, dinv, W_conv, b_conv[None, :])
    h_out = h.transpose(1, 0, 2).reshape(NPAD, D)
    return (marg[:N], mp[:N, 0], h_out[:N])
